# baseline (speedup 1.0000x reference)
import functools

import jax
import jax.numpy as jnp
from jax import lax
from jax.experimental import pallas as pl
from jax.experimental.pallas import tpu as pltpu
from jax.experimental.pallas import tpu_sc as plsc

B = 16384
V = 1000000
CH = 32768
NBLK = (V + CH - 1) // CH
D = 16
L = 16
NC = 2
NS = 16
NW = NC * NS
BPW = B // NW
NCHUNK = BPW // L


def _flatten_body(in_ref, out_ref):
    for d in range(D):
        out_ref[pl.ds(d * CH, CH)] = in_ref[d]


def _flatten_dmajor(table):
    return pl.pallas_call(
        _flatten_body,
        grid=(NBLK,),
        in_specs=[pl.BlockSpec((D, CH), lambda i: (0, i))],
        out_specs=pl.BlockSpec((D * CH,), lambda i: (i,)),
        out_shape=jax.ShapeDtypeStruct((NBLK * D * CH,), jnp.float32),
    )(table.T)


def _sc_body(uid_hbm, iid_hbm, pf_hbm, qf_hbm, al_hbm, be_hbm,
             out_hbm,
             idx_u, idx_i, idx_pf, idx_qf, pu_c, qi_c, a_v, b_v, out_v,
             sem):
    wid = lax.axis_index("s") * NC + lax.axis_index("c")
    base = wid * BPW

    pltpu.sync_copy(uid_hbm.at[pl.ds(base, BPW)], idx_u)
    pltpu.sync_copy(iid_hbm.at[pl.ds(base, BPW)], idx_i)

    def _fix(c, carry):
        o = c * L
        u = jnp.maximum(idx_u[pl.ds(o, L)] - 1, 0)
        idx_u[pl.ds(o, L)] = u
        v = jnp.maximum(idx_i[pl.ds(o, L)] - 1, 0)
        idx_i[pl.ds(o, L)] = v
        ub = ((u >> 15) << 19) + (u & (CH - 1))
        vb = ((v >> 15) << 19) + (v & (CH - 1))
        for d in range(D):
            idx_pf[pl.ds(d * BPW + o, L)] = ub + (d << 15)
            idx_qf[pl.ds(d * BPW + o, L)] = vb + (d << 15)
        return carry

    lax.fori_loop(0, NCHUNK, _fix, 0)

    copies = [
        pltpu.async_copy(pf_hbm.at[idx_pf], pu_c, sem),
        pltpu.async_copy(qf_hbm.at[idx_qf], qi_c, sem),
        pltpu.async_copy(al_hbm.at[idx_u], a_v, sem),
        pltpu.async_copy(be_hbm.at[idx_i], b_v, sem),
    ]
    for cp in copies:
        cp.wait()

    def _compute(c, carry):
        o = c * L
        acc = a_v[pl.ds(o, L)] + b_v[pl.ds(o, L)]
        for d in range(D):
            pu = pu_c[pl.ds(d * BPW + o, L)]
            qi = qi_c[pl.ds(d * BPW + o, L)]
            acc = acc + (pu * qi) * 0.8
        out_v[pl.ds(o, L)] = acc
        return carry

    lax.fori_loop(0, NCHUNK, _compute, 0)

    pltpu.sync_copy(out_v, out_hbm.at[pl.ds(base, BPW)])


_sc_call = functools.partial(
    pl.kernel,
    out_type=jax.ShapeDtypeStruct((B,), jnp.float32),
    mesh=plsc.VectorSubcoreMesh(core_axis_name="c", subcore_axis_name="s"),
    compiler_params=pltpu.CompilerParams(use_tc_tiling_on_sc=False),
    scratch_types=[
        pltpu.VMEM((BPW,), jnp.int32),
        pltpu.VMEM((BPW,), jnp.int32),
        pltpu.VMEM((BPW * D,), jnp.int32),
        pltpu.VMEM((BPW * D,), jnp.int32),
        pltpu.VMEM((BPW * D,), jnp.float32),
        pltpu.VMEM((BPW * D,), jnp.float32),
        pltpu.VMEM((BPW,), jnp.float32),
        pltpu.VMEM((BPW,), jnp.float32),
        pltpu.VMEM((BPW,), jnp.float32),
        pltpu.SemaphoreType.DMA,
    ],
)(_sc_body)


def kernel(sparse_inputs, p, q, user_bias, item_bias, beta_u, bias_u,
           beta_i, bias_i, user_weight, item_weight, user_hs, item_hs,
           u_avg, i_avg):
    uid_raw = sparse_inputs[:, 0]
    iid_raw = sparse_inputs[:, 1]
    alpha = (0.8 * user_bias.T
             + 0.2 * user_weight.T * (beta_u.T * user_hs[None, :]
                                      + bias_u.T)
             + u_avg[None, :]).reshape(-1)
    beta = (0.8 * item_bias.T
            + 0.2 * item_weight.T * (beta_i.T * item_hs[None, :]
                                     + bias_i.T)
            + i_avg[None, :]).reshape(-1)
    pf = _flatten_dmajor(p)
    qf = _flatten_dmajor(q)
    out = _sc_call(uid_raw, iid_raw, pf, qf, alpha, beta)
    return out.reshape(B, 1)


# speedup vs baseline: 1.8080x; 1.8080x over previous
import functools

import jax
import jax.numpy as jnp
from jax import lax
from jax.experimental import pallas as pl
from jax.experimental.pallas import tpu as pltpu
from jax.experimental.pallas import tpu_sc as plsc

B = 16384
V = 1000000
CH = 32768
NBLK = (V + CH - 1) // CH
D = 16
L = 16
NC = 2
NS = 16
NW = NC * NS
BPW = B // NW
NCHUNK = BPW // L


def _flatten_body(in_ref, out_ref):
    for d in range(D):
        out_ref[pl.ds(d * CH, CH)] = in_ref[d]


def _flatten_dmajor(table):
    return pl.pallas_call(
        _flatten_body,
        grid=(NBLK,),
        in_specs=[pl.BlockSpec((D, CH), lambda i: (0, i))],
        out_specs=pl.BlockSpec((D * CH,), lambda i: (i,)),
        out_shape=jax.ShapeDtypeStruct((NBLK * D * CH,), jnp.float32),
    )(table.T)


def _sc_mf_body(uid_hbm, iid_hbm, pf_hbm, qf_hbm,
                mf_hbm,
                idx_u, idx_i, idx_pf, idx_qf, pu_c, qi_c, mf_v,
                sem):
    wid = lax.axis_index("s") * NC + lax.axis_index("c")
    base = wid * BPW

    pltpu.sync_copy(uid_hbm.at[pl.ds(base, BPW)], idx_u)
    pltpu.sync_copy(iid_hbm.at[pl.ds(base, BPW)], idx_i)

    def _fix(c, carry):
        o = c * L
        u = jnp.maximum(idx_u[pl.ds(o, L)] - 1, 0)
        v = jnp.maximum(idx_i[pl.ds(o, L)] - 1, 0)
        ub = ((u >> 15) << 19) + (u & (CH - 1))
        vb = ((v >> 15) << 19) + (v & (CH - 1))
        for d in range(D):
            idx_pf[pl.ds(d * BPW + o, L)] = ub + (d << 15)
            idx_qf[pl.ds(d * BPW + o, L)] = vb + (d << 15)
        return carry

    lax.fori_loop(0, NCHUNK, _fix, 0)

    copies = [
        pltpu.async_copy(pf_hbm.at[idx_pf], pu_c, sem),
        pltpu.async_copy(qf_hbm.at[idx_qf], qi_c, sem),
    ]
    for cp in copies:
        cp.wait()

    def _compute(c, carry):
        o = c * L
        acc = jnp.zeros((L,), jnp.float32)
        for d in range(D):
            pu = pu_c[pl.ds(d * BPW + o, L)]
            qi = qi_c[pl.ds(d * BPW + o, L)]
            acc = acc + pu * qi
        mf_v[pl.ds(o, L)] = acc * 0.8
        return carry

    lax.fori_loop(0, NCHUNK, _compute, 0)

    pltpu.sync_copy(mf_v, mf_hbm.at[pl.ds(base, BPW)])


_sc_mf_call = functools.partial(
    pl.kernel,
    out_type=jax.ShapeDtypeStruct((B,), jnp.float32),
    mesh=plsc.VectorSubcoreMesh(core_axis_name="c", subcore_axis_name="s"),
    compiler_params=pltpu.CompilerParams(use_tc_tiling_on_sc=False),
    scratch_types=[
        pltpu.VMEM((BPW,), jnp.int32),
        pltpu.VMEM((BPW,), jnp.int32),
        pltpu.VMEM((BPW * D,), jnp.int32),
        pltpu.VMEM((BPW * D,), jnp.int32),
        pltpu.VMEM((BPW * D,), jnp.float32),
        pltpu.VMEM((BPW * D,), jnp.float32),
        pltpu.VMEM((BPW,), jnp.float32),
        pltpu.SemaphoreType.DMA,
    ],
)(_sc_mf_body)


def _sc_combine_body(uid_hbm, iid_hbm, al_hbm, be_hbm, mf_hbm,
                     out_hbm,
                     idx_u, idx_i, a_v, b_v, mf_v, out_v, sem):
    wid = lax.axis_index("s") * NC + lax.axis_index("c")
    base = wid * BPW

    pltpu.sync_copy(uid_hbm.at[pl.ds(base, BPW)], idx_u)
    pltpu.sync_copy(iid_hbm.at[pl.ds(base, BPW)], idx_i)
    pltpu.sync_copy(mf_hbm.at[pl.ds(base, BPW)], mf_v)

    def _fix(c, carry):
        o = c * L
        idx_u[pl.ds(o, L)] = jnp.maximum(idx_u[pl.ds(o, L)] - 1, 0)
        idx_i[pl.ds(o, L)] = jnp.maximum(idx_i[pl.ds(o, L)] - 1, 0)
        return carry

    lax.fori_loop(0, NCHUNK, _fix, 0)

    copies = [
        pltpu.async_copy(al_hbm.at[idx_u], a_v, sem),
        pltpu.async_copy(be_hbm.at[idx_i], b_v, sem),
    ]
    for cp in copies:
        cp.wait()

    def _combine(c, carry):
        o = c * L
        out_v[pl.ds(o, L)] = (mf_v[pl.ds(o, L)] + a_v[pl.ds(o, L)]
                              + b_v[pl.ds(o, L)])
        return carry

    lax.fori_loop(0, NCHUNK, _combine, 0)

    pltpu.sync_copy(out_v, out_hbm.at[pl.ds(base, BPW)])


_sc_combine_call = functools.partial(
    pl.kernel,
    out_type=jax.ShapeDtypeStruct((B,), jnp.float32),
    mesh=plsc.VectorSubcoreMesh(core_axis_name="c", subcore_axis_name="s"),
    compiler_params=pltpu.CompilerParams(use_tc_tiling_on_sc=False),
    scratch_types=[
        pltpu.VMEM((BPW,), jnp.int32),
        pltpu.VMEM((BPW,), jnp.int32),
        pltpu.VMEM((BPW,), jnp.float32),
        pltpu.VMEM((BPW,), jnp.float32),
        pltpu.VMEM((BPW,), jnp.float32),
        pltpu.VMEM((BPW,), jnp.float32),
        pltpu.SemaphoreType.DMA,
    ],
)(_sc_combine_body)


def kernel(sparse_inputs, p, q, user_bias, item_bias, beta_u, bias_u,
           beta_i, bias_i, user_weight, item_weight, user_hs, item_hs,
           u_avg, i_avg):
    uid_raw = sparse_inputs[:, 0]
    iid_raw = sparse_inputs[:, 1]
    alpha = (0.8 * user_bias
             + 0.2 * user_weight * (beta_u * user_hs[:, None] + bias_u)
             + u_avg[:, None])[:, 0]
    beta = (0.8 * item_bias
            + 0.2 * item_weight * (beta_i * item_hs[:, None] + bias_i)
            + i_avg[:, None])[:, 0]
    pf = _flatten_dmajor(p)
    qf = _flatten_dmajor(q)
    mf = _sc_mf_call(uid_raw, iid_raw, pf, qf)
    out = _sc_combine_call(uid_raw, iid_raw, alpha, beta, mf)
    return out.reshape(B, 1)


# speedup vs baseline: 1.8735x; 1.0362x over previous
import functools

import jax
import jax.numpy as jnp
from jax import lax
from jax.experimental import pallas as pl
from jax.experimental.pallas import tpu as pltpu
from jax.experimental.pallas import tpu_sc as plsc

B = 16384
V = 1000000
CH = 65536
SH = 16
NBLK = (V + CH - 1) // CH
D = 16
L = 16
NC = 2
NS = 16
NW = NC * NS
BPW = B // NW
NCHUNK = BPW // L


def _flatten_body(in_ref, out_ref):
    for d in range(D):
        out_ref[pl.ds(d * CH, CH)] = in_ref[d]


def _flatten_dmajor(table):
    return pl.pallas_call(
        _flatten_body,
        grid=(NBLK,),
        in_specs=[pl.BlockSpec((D, CH), lambda i: (0, i))],
        out_specs=pl.BlockSpec((D * CH,), lambda i: (i,)),
        out_shape=jax.ShapeDtypeStruct((NBLK * D * CH,), jnp.float32),
    )(table.T)


def _sc_mf_body(uid_hbm, iid_hbm, pf_hbm, qf_hbm,
                mf_hbm,
                idx_u, idx_i, idx_pf, idx_qf, pu_c, qi_c, mf_v,
                sem):
    wid = lax.axis_index("s") * NC + lax.axis_index("c")
    base = wid * BPW

    pltpu.sync_copy(uid_hbm.at[pl.ds(base, BPW)], idx_u)
    pltpu.sync_copy(iid_hbm.at[pl.ds(base, BPW)], idx_i)

    def _fix(c, carry):
        o = c * L
        u = jnp.maximum(idx_u[pl.ds(o, L)] - 1, 0)
        v = jnp.maximum(idx_i[pl.ds(o, L)] - 1, 0)
        ub = ((u >> SH) << (SH + 4)) + (u & (CH - 1))
        vb = ((v >> SH) << (SH + 4)) + (v & (CH - 1))
        for d in range(D):
            idx_pf[pl.ds(d * BPW + o, L)] = ub + (d << SH)
            idx_qf[pl.ds(d * BPW + o, L)] = vb + (d << SH)
        return carry

    lax.fori_loop(0, NCHUNK, _fix, 0)

    copies = [
        pltpu.async_copy(pf_hbm.at[idx_pf], pu_c, sem),
        pltpu.async_copy(qf_hbm.at[idx_qf], qi_c, sem),
    ]
    for cp in copies:
        cp.wait()

    def _compute(c, carry):
        o = c * L
        acc = jnp.zeros((L,), jnp.float32)
        for d in range(D):
            pu = pu_c[pl.ds(d * BPW + o, L)]
            qi = qi_c[pl.ds(d * BPW + o, L)]
            acc = acc + pu * qi
        mf_v[pl.ds(o, L)] = acc * 0.8
        return carry

    lax.fori_loop(0, NCHUNK, _compute, 0)

    pltpu.sync_copy(mf_v, mf_hbm.at[pl.ds(base, BPW)])


_sc_mf_call = functools.partial(
    pl.kernel,
    out_type=jax.ShapeDtypeStruct((B,), jnp.float32),
    mesh=plsc.VectorSubcoreMesh(core_axis_name="c", subcore_axis_name="s"),
    compiler_params=pltpu.CompilerParams(use_tc_tiling_on_sc=False),
    scratch_types=[
        pltpu.VMEM((BPW,), jnp.int32),
        pltpu.VMEM((BPW,), jnp.int32),
        pltpu.VMEM((BPW * D,), jnp.int32),
        pltpu.VMEM((BPW * D,), jnp.int32),
        pltpu.VMEM((BPW * D,), jnp.float32),
        pltpu.VMEM((BPW * D,), jnp.float32),
        pltpu.VMEM((BPW,), jnp.float32),
        pltpu.SemaphoreType.DMA,
    ],
)(_sc_mf_body)


def _sc_combine_body(uid_hbm, iid_hbm, al_hbm, be_hbm, mf_hbm,
                     out_hbm,
                     idx_u, idx_i, a_v, b_v, mf_v, out_v, sem):
    wid = lax.axis_index("s") * NC + lax.axis_index("c")
    base = wid * BPW

    pltpu.sync_copy(uid_hbm.at[pl.ds(base, BPW)], idx_u)
    pltpu.sync_copy(iid_hbm.at[pl.ds(base, BPW)], idx_i)
    pltpu.sync_copy(mf_hbm.at[pl.ds(base, BPW)], mf_v)

    def _fix(c, carry):
        o = c * L
        idx_u[pl.ds(o, L)] = jnp.maximum(idx_u[pl.ds(o, L)] - 1, 0)
        idx_i[pl.ds(o, L)] = jnp.maximum(idx_i[pl.ds(o, L)] - 1, 0)
        return carry

    lax.fori_loop(0, NCHUNK, _fix, 0)

    copies = [
        pltpu.async_copy(al_hbm.at[idx_u], a_v, sem),
        pltpu.async_copy(be_hbm.at[idx_i], b_v, sem),
    ]
    for cp in copies:
        cp.wait()

    def _combine(c, carry):
        o = c * L
        out_v[pl.ds(o, L)] = (mf_v[pl.ds(o, L)] + a_v[pl.ds(o, L)]
                              + b_v[pl.ds(o, L)])
        return carry

    lax.fori_loop(0, NCHUNK, _combine, 0)

    pltpu.sync_copy(out_v, out_hbm.at[pl.ds(base, BPW)])


_sc_combine_call = functools.partial(
    pl.kernel,
    out_type=jax.ShapeDtypeStruct((B,), jnp.float32),
    mesh=plsc.VectorSubcoreMesh(core_axis_name="c", subcore_axis_name="s"),
    compiler_params=pltpu.CompilerParams(use_tc_tiling_on_sc=False),
    scratch_types=[
        pltpu.VMEM((BPW,), jnp.int32),
        pltpu.VMEM((BPW,), jnp.int32),
        pltpu.VMEM((BPW,), jnp.float32),
        pltpu.VMEM((BPW,), jnp.float32),
        pltpu.VMEM((BPW,), jnp.float32),
        pltpu.VMEM((BPW,), jnp.float32),
        pltpu.SemaphoreType.DMA,
    ],
)(_sc_combine_body)


def kernel(sparse_inputs, p, q, user_bias, item_bias, beta_u, bias_u,
           beta_i, bias_i, user_weight, item_weight, user_hs, item_hs,
           u_avg, i_avg):
    uid_raw = sparse_inputs[:, 0]
    iid_raw = sparse_inputs[:, 1]
    alpha = (0.8 * user_bias
             + 0.2 * user_weight * (beta_u * user_hs[:, None] + bias_u)
             + u_avg[:, None])[:, 0]
    beta = (0.8 * item_bias
            + 0.2 * item_weight * (beta_i * item_hs[:, None] + bias_i)
            + i_avg[:, None])[:, 0]
    pf = _flatten_dmajor(p)
    qf = _flatten_dmajor(q)
    mf = _sc_mf_call(uid_raw, iid_raw, pf, qf)
    out = _sc_combine_call(uid_raw, iid_raw, alpha, beta, mf)
    return out.reshape(B, 1)
